# baseline (speedup 1.0000x reference)
import jax
import jax.numpy as jnp
import numpy as np
from jax.experimental import pallas as pl
from jax.experimental.pallas import tpu as pltpu

N_NODES = 10000
N_PAD = 10240
N_EDGES = 320000
NODE_DIM = 128
NUM_BASIS = 20
NB_PAD = 32
N_LAYERS = 3
N_GRAPHS = 64
CUTOFF = 5.0

EC = 1280
EG = N_EDGES // EC
NC = 2048
NG = N_PAD // NC
NSUB = 512
NBLK = N_PAD // NSUB

_F32 = jnp.float32


def _gather_rows(idx_col, table_ref):
    C = idx_col.shape[0]
    acc = jnp.zeros((C, NODE_DIM), _F32)
    for b in range(NBLK):
        ids = jax.lax.broadcasted_iota(jnp.int32, (C, NSUB), 1) + b * NSUB
        oh = (idx_col == ids).astype(_F32)
        acc = acc + jnp.dot(oh, table_ref[b * NSUB:(b + 1) * NSUB, :],
                            preferred_element_type=_F32)
    return acc


def _geom_kernel(src_ref, dst_ref, pos_ref, dist_ref, rx_ref, ry_ref, rz_ref):
    src = src_ref[0]
    dst = dst_ref[0]
    ps = _gather_rows(src, pos_ref)
    pd = _gather_rows(dst, pos_ref)
    vec = pd - ps
    d2 = jnp.sum(vec * vec, axis=1, keepdims=True)
    dist = jnp.sqrt(d2 + 1e-8)
    dist_ref[0] = dist
    rx_ref[0] = vec[:, 0:1] / dist
    ry_ref[0] = vec[:, 1:2] / dist
    rz_ref[0] = vec[:, 2:3] / dist


def _geometry(src_t, dst_t, pos_pad):
    out_shape = [jax.ShapeDtypeStruct((EG, EC, 1), _F32)] * 4
    idx_spec = pl.BlockSpec((1, EC, 1), lambda c: (c, 0, 0))
    col_spec = pl.BlockSpec((1, EC, 1), lambda c: (c, 0, 0))
    return pl.pallas_call(
        _geom_kernel,
        grid=(EG,),
        in_specs=[idx_spec, idx_spec,
                  pl.BlockSpec((N_PAD, NODE_DIM), lambda c: (0, 0))],
        out_specs=[col_spec] * 4,
        out_shape=out_shape,
    )(src_t, dst_t, pos_pad)


def _embed_kernel(at_ref, tab_ref, out_ref):
    idx = at_ref[0]
    ids = jax.lax.broadcasted_iota(jnp.int32, (NC, 128), 1)
    oh = (idx == ids).astype(_F32)
    out_ref[...] = jnp.dot(oh, tab_ref[...], preferred_element_type=_F32)


def _embed(at_t, tab_pad):
    return pl.pallas_call(
        _embed_kernel,
        grid=(NG,),
        in_specs=[pl.BlockSpec((1, NC, 1), lambda c: (c, 0, 0)),
                  pl.BlockSpec((128, NODE_DIM), lambda c: (0, 0))],
        out_specs=pl.BlockSpec((NC, NODE_DIM), lambda c: (c, 0)),
        out_shape=jax.ShapeDtypeStruct((N_PAD, NODE_DIM), _F32),
    )(at_t, tab_pad)


def _proj_kernel(x_ref, wa_ref, wb_ref, h_ref, s_ref):
    x = x_ref[...]
    h_ref[...] = jnp.dot(x, wa_ref[...], preferred_element_type=_F32)
    s_ref[...] = jnp.dot(x, wb_ref[...], preferred_element_type=_F32)


def _proj(x, wa, wb):
    return pl.pallas_call(
        _proj_kernel,
        grid=(NG,),
        in_specs=[pl.BlockSpec((NC, NODE_DIM), lambda c: (c, 0)),
                  pl.BlockSpec((NODE_DIM, NODE_DIM), lambda c: (0, 0)),
                  pl.BlockSpec((NODE_DIM, NODE_DIM), lambda c: (0, 0))],
        out_specs=[pl.BlockSpec((NC, NODE_DIM), lambda c: (c, 0))] * 2,
        out_shape=[jax.ShapeDtypeStruct((N_PAD, NODE_DIM), _F32)] * 2,
    )(x, wa, wb)


def _scatter_kernel(src_ref, dstv_ref, dsts_ref, dist_ref, rx_ref, ry_ref,
                    rz_ref, h_ref, wrbf_ref, agg_ref, m4_ref):
    c = pl.program_id(0)

    @pl.when(c == 0)
    def _init():
        agg_ref[...] = jnp.zeros_like(agg_ref)

    dist = dist_ref[0]
    gamma = (NUM_BASIS / CUTOFF) ** 2
    step = CUTOFF / (NUM_BASIS - 1)
    centers = jax.lax.broadcasted_iota(jnp.int32, (EC, NB_PAD), 1).astype(_F32) * step
    ea = jnp.exp(-gamma * (dist - centers) ** 2)
    r = jnp.clip(dist / CUTOFF, 0.0, 1.0)
    fcut = 0.5 * (jnp.cos(jnp.pi * r) + 1.0)
    ea = ea * fcut
    filt = jnp.dot(ea, wrbf_ref[...], preferred_element_type=_F32)

    hs = _gather_rows(src_ref[0], h_ref)
    m = hs * filt
    m4 = jnp.concatenate(
        [m, m * rx_ref[0], m * ry_ref[0], m * rz_ref[0]], axis=1)
    m4_ref[...] = m4

    def body(i, _):
        d = dsts_ref[0, 0, i]
        row = m4_ref[pl.ds(i, 1), :]
        agg_ref[pl.ds(d, 1), :] = agg_ref[pl.ds(d, 1), :] + row
        return 0

    jax.lax.fori_loop(0, EC, body, 0, unroll=False)


def _scatter(src_t, dst_t, dst_s, dist, rx, ry, rz, h, wrbf_pad):
    idx_spec = pl.BlockSpec((1, EC, 1), lambda c: (c, 0, 0))
    smem_spec = pl.BlockSpec((1, 1, EC), lambda c: (c, 0, 0),
                             memory_space=pltpu.SMEM)
    col_spec = pl.BlockSpec((1, EC, 1), lambda c: (c, 0, 0))
    return pl.pallas_call(
        _scatter_kernel,
        grid=(EG,),
        in_specs=[idx_spec, idx_spec, smem_spec, col_spec, col_spec, col_spec,
                  col_spec,
                  pl.BlockSpec((N_PAD, NODE_DIM), lambda c: (0, 0)),
                  pl.BlockSpec((NB_PAD, NODE_DIM), lambda c: (0, 0))],
        out_specs=pl.BlockSpec((N_PAD, 4 * NODE_DIM), lambda c: (0, 0)),
        out_shape=jax.ShapeDtypeStruct((N_PAD, 4 * NODE_DIM), _F32),
        scratch_shapes=[pltpu.VMEM((EC, 4 * NODE_DIM), _F32)],
    )(src_t, dst_t, dst_s, dist, rx, ry, rz, h, wrbf_pad)


def _update_kernel(agg_ref, self_ref, out_ref):
    a = agg_ref[...]
    agg = a[:, 0:128]
    vx = a[:, 128:256]
    vy = a[:, 256:384]
    vz = a[:, 384:512]
    vnorm = jnp.sqrt(vx * vx + vy * vy + vz * vz + 1e-8)
    pre = agg + vnorm + self_ref[...]
    out_ref[...] = pre * jax.nn.sigmoid(pre)


def _update(agg4, selfterm):
    return pl.pallas_call(
        _update_kernel,
        grid=(NG,),
        in_specs=[pl.BlockSpec((NC, 4 * NODE_DIM), lambda c: (c, 0)),
                  pl.BlockSpec((NC, NODE_DIM), lambda c: (c, 0))],
        out_specs=pl.BlockSpec((NC, NODE_DIM), lambda c: (c, 0)),
        out_shape=jax.ShapeDtypeStruct((N_PAD, NODE_DIM), _F32),
    )(agg4, selfterm)


def _head_kernel(x_ref, wout_ref, batch_ref, out_ref):
    c = pl.program_id(0)

    @pl.when(c == 0)
    def _init():
        out_ref[...] = jnp.zeros_like(out_ref)

    no = jnp.dot(x_ref[...], wout_ref[...], preferred_element_type=_F32)
    g_ids = jax.lax.broadcasted_iota(jnp.int32, (N_GRAPHS, NC), 0)
    oh = (g_ids == batch_ref[0]).astype(_F32)
    out_ref[...] = out_ref[...] + jnp.dot(oh, no, preferred_element_type=_F32)


def _head(x, wout_pad, batch_r):
    return pl.pallas_call(
        _head_kernel,
        grid=(NG,),
        in_specs=[pl.BlockSpec((NC, NODE_DIM), lambda c: (c, 0)),
                  pl.BlockSpec((NODE_DIM, NODE_DIM), lambda c: (0, 0)),
                  pl.BlockSpec((1, 1, NC), lambda c: (c, 0, 0))],
        out_specs=pl.BlockSpec((N_GRAPHS, NODE_DIM), lambda c: (0, 0)),
        out_shape=jax.ShapeDtypeStruct((N_GRAPHS, NODE_DIM), _F32),
    )(x, wout_pad, batch_r)


def kernel(at_no, pos, edge_index, batch, embed_table, W_src, W_rbf, W_self, W_out):
    src_t = edge_index[0].astype(jnp.int32).reshape(EG, EC, 1)
    dst_t = edge_index[1].astype(jnp.int32).reshape(EG, EC, 1)
    dst_s = edge_index[1].astype(jnp.int32).reshape(EG, 1, EC)
    pos_pad = jnp.zeros((N_PAD, NODE_DIM), _F32).at[:N_NODES, :3].set(pos)
    at_t = jnp.full((N_PAD,), 0, jnp.int32).at[:N_NODES].set(
        at_no.astype(jnp.int32)).reshape(NG, NC, 1)
    batch_r = jnp.full((N_PAD,), N_GRAPHS, jnp.int32).at[:N_NODES].set(
        batch.astype(jnp.int32)).reshape(NG, 1, NC)
    tab_pad = jnp.zeros((128, NODE_DIM), _F32).at[:embed_table.shape[0]].set(
        embed_table)
    wrbf_pad = jnp.zeros((N_LAYERS, NB_PAD, NODE_DIM), _F32).at[:, :NUM_BASIS].set(
        W_rbf)
    wout_pad = jnp.zeros((NODE_DIM, NODE_DIM), _F32).at[:, :1].set(W_out)

    dist, rx, ry, rz = _geometry(src_t, dst_t, pos_pad)
    x = _embed(at_t, tab_pad)
    for l in range(N_LAYERS):
        h, selfterm = _proj(x, W_src[l], W_self[l])
        agg4 = _scatter(src_t, dst_t, dst_s, dist, rx, ry, rz, h, wrbf_pad[l])
        x = _update(agg4, selfterm)
    out = _head(x, wout_pad, batch_r)
    return out[:, :1]
